# baseline (speedup 1.0000x reference)
import functools

import jax
import jax.numpy as jnp
from jax import lax
from jax.experimental import pallas as pl
from jax.experimental.pallas import tpu as pltpu
from jax.experimental.pallas import tpu_sc as plsc

N = 10000
E = 320000
D = 128
NC = 2
NS = 16
NW = NC * NS
K = 128
CH = 80
EPT = CH * K
E_PAD = NW * EPT
NP = 10240
ROWS_PER_TILE = NP // NS
DROWS = NP // D


_MESH = plsc.VectorSubcoreMesh(core_axis_name="c", subcore_axis_name="s",
                               num_cores=NC, num_subcores=NS)


def _sc_agg_body(with_deg, x_hbm, src_hbm, dst_hbm, agg_out, *rest):
    if with_deg:
        (deg_out, src_v, dst_v, buf, ones_v, agg_sh, deg_sh, sem) = rest
    else:
        deg_out = ones_v = deg_sh = None
        (src_v, dst_v, buf, agg_sh, sem) = rest

    c = lax.axis_index("c")
    s = lax.axis_index("s")
    w = c * NS + s

    zeros16 = jnp.zeros((16,), jnp.float32)

    def _zero_buf_row(r, carry):
        for j in range(D // 16):
            buf[r, pl.ds(j * 16, 16)] = zeros16
        return carry
    lax.fori_loop(0, K, _zero_buf_row, 0)

    base = s * ROWS_PER_TILE

    def _init(i, carry):
        pltpu.sync_copy(buf, agg_sh.at[pl.ds(base + i * K, K), :])
        return carry
    lax.fori_loop(0, ROWS_PER_TILE // K, _init, 0)

    if with_deg:
        def _zfill(r, carry):
            ones_v[pl.ds(r * 16, 16)] = zeros16
            return carry
        lax.fori_loop(0, ROWS_PER_TILE // 16, _zfill, 0)
        pltpu.sync_copy(ones_v.at[pl.ds(0, ROWS_PER_TILE)],
                        deg_sh.at[pl.ds(base, ROWS_PER_TILE)])
        ones16 = jnp.full((16,), 1.0, jnp.float32)

        def _ofill(r, carry):
            ones_v[pl.ds(r * 16, 16)] = ones16
            return carry
        lax.fori_loop(0, K // 16, _ofill, 0)

    pltpu.sync_copy(src_hbm.at[w], src_v)
    pltpu.sync_copy(dst_hbm.at[w], dst_v)

    plsc.subcore_barrier()

    def _chunk(g, carry):
        pltpu.async_copy(x_hbm.at[src_v.at[g]], buf, sem).wait()
        if with_deg:
            pltpu.sync_copy(ones_v.at[pl.ds(0, K)],
                            deg_sh.at[dst_v.at[g]], add=True)
        pltpu.sync_copy(buf, agg_sh.at[dst_v.at[g]], add=True)
        return carry
    lax.fori_loop(0, CH, _chunk, 0)

    plsc.subcore_barrier()

    def _copy_out(i, carry):
        pltpu.sync_copy(agg_sh.at[pl.ds(base + i * K, K), :],
                        agg_out.at[c, pl.ds(base + i * K, K), :])
        return carry
    lax.fori_loop(0, ROWS_PER_TILE // K, _copy_out, 0)
    if with_deg:
        pltpu.sync_copy(deg_sh.at[pl.ds(base, ROWS_PER_TILE)],
                        deg_out.at[c, pl.ds(base, ROWS_PER_TILE)])


def _make_sc_agg(with_deg):
    out_type = [jax.ShapeDtypeStruct((NC, NP, D), jnp.float32)]
    scratch = [
        pltpu.VMEM((CH, K), jnp.int32),
        pltpu.VMEM((CH, K), jnp.int32),
        pltpu.VMEM((K, D), jnp.float32),
    ]
    if with_deg:
        out_type.append(jax.ShapeDtypeStruct((NC, NP), jnp.float32))
        scratch.append(pltpu.VMEM((ROWS_PER_TILE,), jnp.float32))
    scratch.append(pltpu.VMEM_SHARED((NP, D), jnp.float32))
    if with_deg:
        scratch.append(pltpu.VMEM_SHARED((NP,), jnp.float32))
    scratch.append(pltpu.SemaphoreType.DMA)

    return pl.kernel(
        functools.partial(_sc_agg_body, with_deg),
        out_type=out_type,
        mesh=_MESH,
        scratch_types=scratch,
        name="sage_agg_deg" if with_deg else "sage_agg",
    )


_sc_agg_deg = _make_sc_agg(True)
_sc_agg = _make_sc_agg(False)


BR = 1024


def _dense_body(final, x_ref, p0, p1, dall, ws, wn, b, *rest):
    if final:
        wo, bo, o_ref = rest
    else:
        (o_ref,) = rest
    deg = jnp.maximum(jnp.sum(dall[...], axis=0), 1.0)
    mean = (p0[...] + p1[...]) / deg
    h = jnp.dot(x_ref[...], ws[...], preferred_element_type=jnp.float32)
    h = h + jnp.dot(mean, wn[...], preferred_element_type=jnp.float32)
    h = jnp.maximum(h + b[...], 0.0)
    if final:
        h = jnp.dot(h, wo[...], preferred_element_type=jnp.float32) + bo[...]
        h = jnp.maximum(h, 0.0)
    o_ref[...] = h


def _make_dense(final):
    row_spec = pl.BlockSpec((BR, D), lambda i: (i, 0))
    deg_spec = pl.BlockSpec((NC, BR, 1), lambda i: (0, i, 0))
    w_spec = pl.BlockSpec((D, D), lambda i: (0, 0))
    b_spec = pl.BlockSpec((1, D), lambda i: (0, 0))
    in_specs = [row_spec, row_spec, row_spec, deg_spec,
                w_spec, w_spec, b_spec]
    if final:
        in_specs += [w_spec, b_spec]
    return pl.pallas_call(
        functools.partial(_dense_body, final),
        grid=(NP // BR,),
        in_specs=in_specs,
        out_specs=row_spec,
        out_shape=jax.ShapeDtypeStruct((NP, D), jnp.float32),
    )


_dense_mid = _make_dense(False)
_dense_final = _make_dense(True)


@jax.jit
def kernel(x, edge_index, W_self1, W_nbr1, b1, W_self2, W_nbr2, b2,
           W_out, b_out):
    src = edge_index[0].astype(jnp.int32)
    dst = edge_index[1].astype(jnp.int32)
    pad = E_PAD - E
    src_p = jnp.concatenate([src, jnp.zeros((pad,), jnp.int32)])
    dst_p = jnp.concatenate([dst, jnp.full((pad,), N, jnp.int32)])
    src_p = src_p.reshape(NW, CH, K)
    dst_p = dst_p.reshape(NW, CH, K)
    x_pad = jnp.pad(x, ((0, NP - N), (0, 0)))

    agg1, deg = _sc_agg_deg(x_pad, src_p, dst_p)
    dall = deg.reshape(NC, NP, 1)

    h1 = _dense_mid(x_pad, agg1[0], agg1[1], dall,
                    W_self1, W_nbr1, b1.reshape(1, D))

    agg2 = _sc_agg(h1, src_p, dst_p)
    if isinstance(agg2, (list, tuple)):
        agg2 = agg2[0]

    out = _dense_final(h1, agg2[0], agg2[1], dall,
                       W_self2, W_nbr2, b2.reshape(1, D),
                       W_out, b_out.reshape(1, D))
    return out[:N]


# speedup vs baseline: 1.0917x; 1.0917x over previous
import functools

import jax
import jax.numpy as jnp
from jax import lax
from jax.experimental import pallas as pl
from jax.experimental.pallas import tpu as pltpu
from jax.experimental.pallas import tpu_sc as plsc

N = 10000
E = 320000
D = 128
NC = 2
NS = 16
NW = NC * NS
K = 128
CH = 80
EPT = CH * K
E_PAD = NW * EPT
NP = 10240
ROWS_PER_TILE = NP // NS
GS = 4
GR = K // GS

_MESH = plsc.VectorSubcoreMesh(core_axis_name="c", subcore_axis_name="s",
                               num_cores=NC, num_subcores=NS)


def _sc_agg_body(x_hbm, srcf_hbm, dst_hbm, agg_out, src_f, dst_v, buf,
                 agg_sh, sem):
    c = lax.axis_index("c")
    s = lax.axis_index("s")
    w = c * NS + s

    zeros16 = jnp.zeros((16,), jnp.float32)

    def _zero_buf_row(r, carry):
        for j in range(D // 16):
            buf[r, pl.ds(j * 16, 16)] = zeros16
        return carry
    lax.fori_loop(0, K, _zero_buf_row, 0)

    base = s * ROWS_PER_TILE

    def _init(i, carry):
        pltpu.sync_copy(buf, agg_sh.at[pl.ds(base + i * K, K), :])
        return carry
    lax.fori_loop(0, ROWS_PER_TILE // K, _init, 0)

    pltpu.sync_copy(srcf_hbm.at[w], src_f)
    pltpu.sync_copy(dst_hbm.at[w], dst_v)

    plsc.subcore_barrier()

    def _chunk(g, carry):
        for t in range(GS):
            pltpu.async_copy(
                x_hbm.at[src_f.at[pl.ds(g * K + t * GR, GR)]],
                buf.at[pl.ds(t * GR, GR), :], sem)
        for t in range(GS):
            pltpu.make_async_copy(
                x_hbm.at[src_f.at[pl.ds(g * K + t * GR, GR)]],
                buf.at[pl.ds(t * GR, GR), :], sem).wait()
        pltpu.sync_copy(buf, agg_sh.at[dst_v.at[g]], add=True)
        return carry
    lax.fori_loop(0, CH, _chunk, 0)

    plsc.subcore_barrier()

    def _copy_out(i, carry):
        pltpu.sync_copy(agg_sh.at[pl.ds(base + i * K, K), :],
                        agg_out.at[c, pl.ds(base + i * K, K), :])
        return carry
    lax.fori_loop(0, ROWS_PER_TILE // K, _copy_out, 0)


_sc_agg = pl.kernel(
    _sc_agg_body,
    out_type=jax.ShapeDtypeStruct((NC, NP, D), jnp.float32),
    mesh=_MESH,
    scratch_types=[
        pltpu.VMEM((EPT,), jnp.int32),
        pltpu.VMEM((CH, K), jnp.int32),
        pltpu.VMEM((K, D), jnp.float32),
        pltpu.VMEM_SHARED((NP, D), jnp.float32),
        pltpu.SemaphoreType.DMA,
    ],
    name="sage_agg",
)


def _sc_deg_body(dst_hbm, deg_out, dst_v, ones_v, deg_sh):
    c = lax.axis_index("c")
    s = lax.axis_index("s")
    w = c * NS + s

    ones16 = jnp.full((16,), 1.0, jnp.float32)
    zeros16 = jnp.zeros((16,), jnp.float32)

    def _zfill(r, carry):
        ones_v[pl.ds(r * 16, 16)] = zeros16
        return carry
    lax.fori_loop(0, ROWS_PER_TILE // 16, _zfill, 0)
    base = s * ROWS_PER_TILE
    pltpu.sync_copy(ones_v.at[pl.ds(0, ROWS_PER_TILE)],
                    deg_sh.at[pl.ds(base, ROWS_PER_TILE)])

    def _ofill(r, carry):
        ones_v[pl.ds(r * 16, 16)] = ones16
        return carry
    lax.fori_loop(0, K // 16, _ofill, 0)

    pltpu.sync_copy(dst_hbm.at[w], dst_v)

    plsc.subcore_barrier()

    def _chunk(g, carry):
        pltpu.sync_copy(ones_v.at[pl.ds(0, K)], deg_sh.at[dst_v.at[g]],
                        add=True)
        return carry
    lax.fori_loop(0, CH, _chunk, 0)

    plsc.subcore_barrier()

    pltpu.sync_copy(deg_sh.at[pl.ds(base, ROWS_PER_TILE)],
                    deg_out.at[c, pl.ds(base, ROWS_PER_TILE)])


_sc_deg = pl.kernel(
    _sc_deg_body,
    out_type=jax.ShapeDtypeStruct((NC, NP), jnp.float32),
    mesh=_MESH,
    scratch_types=[
        pltpu.VMEM((CH, K), jnp.int32),
        pltpu.VMEM((ROWS_PER_TILE,), jnp.float32),
        pltpu.VMEM_SHARED((NP,), jnp.float32),
    ],
    name="sage_deg",
)


BR = 1024


def _dense_body(final, x_ref, p0, p1, dall, ws, wn, b, *rest):
    if final:
        wo, bo, o_ref = rest
    else:
        (o_ref,) = rest
    deg = jnp.maximum(jnp.sum(dall[...], axis=0), 1.0)
    mean = (p0[...] + p1[...]) / deg
    h = jnp.dot(x_ref[...], ws[...], preferred_element_type=jnp.float32)
    h = h + jnp.dot(mean, wn[...], preferred_element_type=jnp.float32)
    h = jnp.maximum(h + b[...], 0.0)
    if final:
        h = jnp.dot(h, wo[...], preferred_element_type=jnp.float32) + bo[...]
        h = jnp.maximum(h, 0.0)
    o_ref[...] = h


def _make_dense(final):
    row_spec = pl.BlockSpec((BR, D), lambda i: (i, 0))
    deg_spec = pl.BlockSpec((NC, BR, 1), lambda i: (0, i, 0))
    w_spec = pl.BlockSpec((D, D), lambda i: (0, 0))
    b_spec = pl.BlockSpec((1, D), lambda i: (0, 0))
    in_specs = [row_spec, row_spec, row_spec, deg_spec,
                w_spec, w_spec, b_spec]
    if final:
        in_specs += [w_spec, b_spec]
    return pl.pallas_call(
        functools.partial(_dense_body, final),
        grid=(NP // BR,),
        in_specs=in_specs,
        out_specs=row_spec,
        out_shape=jax.ShapeDtypeStruct((NP, D), jnp.float32),
    )


_dense_mid = _make_dense(False)
_dense_final = _make_dense(True)


@jax.jit
def kernel(x, edge_index, W_self1, W_nbr1, b1, W_self2, W_nbr2, b2,
           W_out, b_out):
    src = edge_index[0].astype(jnp.int32)
    dst = edge_index[1].astype(jnp.int32)
    pad = E_PAD - E
    src_f = jnp.concatenate([src, jnp.zeros((pad,), jnp.int32)])
    dst_p = jnp.concatenate([dst, jnp.full((pad,), N, jnp.int32)])
    src_f = src_f.reshape(NW, EPT)
    dst_p = dst_p.reshape(NW, CH, K)
    x_pad = jnp.pad(x, ((0, NP - N), (0, 0)))

    dall = _sc_deg(dst_p).reshape(NC, NP, 1)
    agg1 = _sc_agg(x_pad, src_f, dst_p)

    h1 = _dense_mid(x_pad, agg1[0], agg1[1], dall,
                    W_self1, W_nbr1, b1.reshape(1, D))

    agg2 = _sc_agg(h1, src_f, dst_p)

    out = _dense_final(h1, agg2[0], agg2[1], dall,
                       W_self2, W_nbr2, b2.reshape(1, D),
                       W_out, b_out.reshape(1, D))
    return out[:N]


# speedup vs baseline: 2.4173x; 2.2144x over previous
import functools

import jax
import jax.numpy as jnp
from jax import lax
from jax.experimental import pallas as pl
from jax.experimental.pallas import tpu as pltpu
from jax.experimental.pallas import tpu_sc as plsc

N = 10000
E = 320000
D = 128
NC = 2
NS = 16
NW = NC * NS
K = 80
CH = 125
EPT = CH * K
NP = 10240
ROWS_PER_TILE = NP // NS

_MESH = plsc.VectorSubcoreMesh(core_axis_name="c", subcore_axis_name="s",
                               num_cores=NC, num_subcores=NS)


def _sc_agg_body(x_hbm, srcf_hbm, dst_hbm, agg_out, src_f, dst_v, buf,
                 agg_sh):
    c = lax.axis_index("c")
    s = lax.axis_index("s")
    w = c * NS + s

    zeros16 = jnp.zeros((16,), jnp.float32)

    def _zero_buf_row(r, carry):
        for j in range(D // 16):
            buf[r, pl.ds(j * 16, 16)] = zeros16
        return carry
    lax.fori_loop(0, K, _zero_buf_row, 0)

    base = s * ROWS_PER_TILE

    def _init(i, carry):
        pltpu.sync_copy(buf, agg_sh.at[pl.ds(base + i * K, K), :])
        return carry
    lax.fori_loop(0, ROWS_PER_TILE // K, _init, 0)

    pltpu.sync_copy(srcf_hbm.at[w], src_f)
    pltpu.sync_copy(dst_hbm.at[w], dst_v)

    plsc.subcore_barrier()

    def _chunk(g, carry):
        pltpu.sync_copy(x_hbm.at[src_f.at[pl.ds(g * K, K)]], buf)
        pltpu.sync_copy(buf, agg_sh.at[dst_v.at[g]], add=True)
        return carry
    lax.fori_loop(0, CH, _chunk, 0)

    plsc.subcore_barrier()

    def _copy_out(i, carry):
        pltpu.sync_copy(agg_sh.at[pl.ds(base + i * K, K), :],
                        agg_out.at[c, pl.ds(base + i * K, K), :])
        return carry
    lax.fori_loop(0, ROWS_PER_TILE // K, _copy_out, 0)


_sc_agg = pl.kernel(
    _sc_agg_body,
    out_type=jax.ShapeDtypeStruct((NC, NP, D), jnp.float32),
    mesh=_MESH,
    scratch_types=[
        pltpu.VMEM((EPT,), jnp.int32),
        pltpu.VMEM((CH, K), jnp.int32),
        pltpu.VMEM((K, D), jnp.float32),
        pltpu.VMEM_SHARED((NP, D), jnp.float32),
    ],
    name="sage_agg",
)


def _sc_deg_body(dst_hbm, deg_out, dst_v, ones_v, deg_sh):
    c = lax.axis_index("c")
    s = lax.axis_index("s")
    w = c * NS + s

    ones16 = jnp.full((16,), 1.0, jnp.float32)
    zeros16 = jnp.zeros((16,), jnp.float32)

    def _zfill(r, carry):
        ones_v[pl.ds(r * 16, 16)] = zeros16
        return carry
    lax.fori_loop(0, ROWS_PER_TILE // 16, _zfill, 0)
    base = s * ROWS_PER_TILE
    pltpu.sync_copy(ones_v.at[pl.ds(0, ROWS_PER_TILE)],
                    deg_sh.at[pl.ds(base, ROWS_PER_TILE)])

    def _ofill(r, carry):
        ones_v[pl.ds(r * 16, 16)] = ones16
        return carry
    lax.fori_loop(0, K // 16, _ofill, 0)

    pltpu.sync_copy(dst_hbm.at[w], dst_v)

    plsc.subcore_barrier()

    def _chunk(g, carry):
        pltpu.sync_copy(ones_v.at[pl.ds(0, K)], deg_sh.at[dst_v.at[g]],
                        add=True)
        return carry
    lax.fori_loop(0, CH, _chunk, 0)

    plsc.subcore_barrier()

    pltpu.sync_copy(deg_sh.at[pl.ds(base, ROWS_PER_TILE)],
                    deg_out.at[c, pl.ds(base, ROWS_PER_TILE)])


_sc_deg = pl.kernel(
    _sc_deg_body,
    out_type=jax.ShapeDtypeStruct((NC, NP), jnp.float32),
    mesh=_MESH,
    scratch_types=[
        pltpu.VMEM((CH, K), jnp.int32),
        pltpu.VMEM((ROWS_PER_TILE,), jnp.float32),
        pltpu.VMEM_SHARED((NP,), jnp.float32),
    ],
    name="sage_deg",
)


BR = 1000


def _dense_body(final, x_ref, p0, p1, dall, ws, wn, b, *rest):
    if final:
        wo, bo, o_ref = rest
    else:
        (o_ref,) = rest
    deg = jnp.maximum(jnp.sum(dall[...], axis=0), 1.0)
    mean = (p0[...] + p1[...]) / deg
    h = jnp.dot(x_ref[...], ws[...], preferred_element_type=jnp.float32)
    h = h + jnp.dot(mean, wn[...], preferred_element_type=jnp.float32)
    h = jnp.maximum(h + b[...], 0.0)
    if final:
        h = jnp.dot(h, wo[...], preferred_element_type=jnp.float32) + bo[...]
        h = jnp.maximum(h, 0.0)
    o_ref[...] = h


def _make_dense(final):
    row_spec = pl.BlockSpec((BR, D), lambda i: (i, 0))
    deg_spec = pl.BlockSpec((NC, BR, 1), lambda i: (0, i, 0))
    w_spec = pl.BlockSpec((D, D), lambda i: (0, 0))
    b_spec = pl.BlockSpec((1, D), lambda i: (0, 0))
    in_specs = [row_spec, row_spec, row_spec, deg_spec,
                w_spec, w_spec, b_spec]
    if final:
        in_specs += [w_spec, b_spec]
    return pl.pallas_call(
        functools.partial(_dense_body, final),
        grid=(N // BR,),
        in_specs=in_specs,
        out_specs=row_spec,
        out_shape=jax.ShapeDtypeStruct((N, D), jnp.float32),
    )


_dense_mid = _make_dense(False)
_dense_final = _make_dense(True)


@jax.jit
def kernel(x, edge_index, W_self1, W_nbr1, b1, W_self2, W_nbr2, b2,
           W_out, b_out):
    src_f = edge_index[0].astype(jnp.int32).reshape(NW, EPT)
    dst_p = edge_index[1].astype(jnp.int32).reshape(NW, CH, K)

    dall = _sc_deg(dst_p).reshape(NC, NP, 1)
    agg1 = _sc_agg(x, src_f, dst_p)

    h1 = _dense_mid(x, agg1[0], agg1[1], dall,
                    W_self1, W_nbr1, b1.reshape(1, D))

    agg2 = _sc_agg(h1, src_f, dst_p)

    out = _dense_final(h1, agg2[0], agg2[1], dall,
                       W_self2, W_nbr2, b2.reshape(1, D),
                       W_out, b_out.reshape(1, D))
    return out


# speedup vs baseline: 3.7228x; 1.5401x over previous
import functools

import jax
import jax.numpy as jnp
from jax import lax
from jax.experimental import pallas as pl
from jax.experimental.pallas import tpu as pltpu
from jax.experimental.pallas import tpu_sc as plsc

N = 10000
E = 320000
D = 128
NC = 2
NS = 16
NW = NC * NS
K = 80
CH = 125
EPT = CH * K
NP = 10240
ROWS_PER_TILE = NP // NS

_MESH = plsc.VectorSubcoreMesh(core_axis_name="c", subcore_axis_name="s",
                               num_cores=NC, num_subcores=NS)


def _sc_agg_body(x_hbm, srcf_hbm, dst_hbm, agg_out, src_f, dst_v, buf,
                 agg_sh, sem0, sem1):
    c = lax.axis_index("c")
    s = lax.axis_index("s")
    w = c * NS + s

    zeros16 = jnp.zeros((16,), jnp.float32)

    def _zero_buf_row(r, carry):
        for j in range(D // 16):
            buf[0, r, pl.ds(j * 16, 16)] = zeros16
        return carry
    lax.fori_loop(0, K, _zero_buf_row, 0)

    base = s * ROWS_PER_TILE

    def _init(i, carry):
        pltpu.sync_copy(buf.at[0], agg_sh.at[pl.ds(base + i * K, K), :])
        return carry
    lax.fori_loop(0, ROWS_PER_TILE // K, _init, 0)

    pltpu.sync_copy(srcf_hbm.at[w], src_f)
    pltpu.sync_copy(dst_hbm.at[w], dst_v)

    plsc.subcore_barrier()

    def _gather(g, slot, sem):
        pltpu.async_copy(x_hbm.at[src_f.at[pl.ds(g * K, K)]],
                         buf.at[slot], sem)

    def _gather_wait(g, slot, sem):
        pltpu.make_async_copy(x_hbm.at[src_f.at[pl.ds(g * K, K)]],
                              buf.at[slot], sem).wait()

    def _scatter(g, slot):
        pltpu.sync_copy(buf.at[slot], agg_sh.at[dst_v.at[g]], add=True)

    _gather(0, 0, sem0)

    def _pair(p, carry):
        e = 2 * p
        _gather(e + 1, 1, sem1)
        _gather_wait(e, 0, sem0)
        _scatter(e, 0)
        _gather(e + 2, 0, sem0)
        _gather_wait(e + 1, 1, sem1)
        _scatter(e + 1, 1)
        return carry
    lax.fori_loop(0, (CH - 1) // 2, _pair, 0)

    _gather_wait(CH - 1, 0, sem0)
    _scatter(CH - 1, 0)

    plsc.subcore_barrier()

    def _copy_out(i, carry):
        pltpu.sync_copy(agg_sh.at[pl.ds(base + i * K, K), :],
                        agg_out.at[c, pl.ds(base + i * K, K), :])
        return carry
    lax.fori_loop(0, ROWS_PER_TILE // K, _copy_out, 0)


_sc_agg = pl.kernel(
    _sc_agg_body,
    out_type=jax.ShapeDtypeStruct((NC, NP, D), jnp.float32),
    mesh=_MESH,
    scratch_types=[
        pltpu.VMEM((EPT,), jnp.int32),
        pltpu.VMEM((CH, K), jnp.int32),
        pltpu.VMEM((2, K, D), jnp.float32),
        pltpu.VMEM_SHARED((NP, D), jnp.float32),
        pltpu.SemaphoreType.DMA,
        pltpu.SemaphoreType.DMA,
    ],
    name="sage_agg",
)


def _sc_deg_body(dst_hbm, deg_out, dst_v, ones_v, deg_sh):
    c = lax.axis_index("c")
    s = lax.axis_index("s")
    w = c * NS + s

    ones16 = jnp.full((16,), 1.0, jnp.float32)
    zeros16 = jnp.zeros((16,), jnp.float32)

    def _zfill(r, carry):
        ones_v[pl.ds(r * 16, 16)] = zeros16
        return carry
    lax.fori_loop(0, ROWS_PER_TILE // 16, _zfill, 0)
    base = s * ROWS_PER_TILE
    pltpu.sync_copy(ones_v.at[pl.ds(0, ROWS_PER_TILE)],
                    deg_sh.at[pl.ds(base, ROWS_PER_TILE)])

    def _ofill(r, carry):
        ones_v[pl.ds(r * 16, 16)] = ones16
        return carry
    lax.fori_loop(0, K // 16, _ofill, 0)

    pltpu.sync_copy(dst_hbm.at[w], dst_v)

    plsc.subcore_barrier()

    def _chunk(g, carry):
        pltpu.sync_copy(ones_v.at[pl.ds(0, K)], deg_sh.at[dst_v.at[g]],
                        add=True)
        return carry
    lax.fori_loop(0, CH, _chunk, 0)

    plsc.subcore_barrier()

    pltpu.sync_copy(deg_sh.at[pl.ds(base, ROWS_PER_TILE)],
                    deg_out.at[c, pl.ds(base, ROWS_PER_TILE)])


_sc_deg = pl.kernel(
    _sc_deg_body,
    out_type=jax.ShapeDtypeStruct((NC, NP), jnp.float32),
    mesh=_MESH,
    scratch_types=[
        pltpu.VMEM((CH, K), jnp.int32),
        pltpu.VMEM((ROWS_PER_TILE,), jnp.float32),
        pltpu.VMEM_SHARED((NP,), jnp.float32),
    ],
    name="sage_deg",
)


BR = 1000


def _dense_body(final, x_ref, p0, p1, dall, ws, wn, b, *rest):
    if final:
        wo, bo, o_ref = rest
    else:
        (o_ref,) = rest
    deg = jnp.maximum(jnp.sum(dall[...], axis=0), 1.0)
    mean = (p0[...] + p1[...]) / deg
    h = jnp.dot(x_ref[...], ws[...], preferred_element_type=jnp.float32)
    h = h + jnp.dot(mean, wn[...], preferred_element_type=jnp.float32)
    h = jnp.maximum(h + b[...], 0.0)
    if final:
        h = jnp.dot(h, wo[...], preferred_element_type=jnp.float32) + bo[...]
        h = jnp.maximum(h, 0.0)
    o_ref[...] = h


def _make_dense(final):
    row_spec = pl.BlockSpec((BR, D), lambda i: (i, 0))
    deg_spec = pl.BlockSpec((NC, BR, 1), lambda i: (0, i, 0))
    w_spec = pl.BlockSpec((D, D), lambda i: (0, 0))
    b_spec = pl.BlockSpec((1, D), lambda i: (0, 0))
    in_specs = [row_spec, row_spec, row_spec, deg_spec,
                w_spec, w_spec, b_spec]
    if final:
        in_specs += [w_spec, b_spec]
    return pl.pallas_call(
        functools.partial(_dense_body, final),
        grid=(N // BR,),
        in_specs=in_specs,
        out_specs=row_spec,
        out_shape=jax.ShapeDtypeStruct((N, D), jnp.float32),
    )


_dense_mid = _make_dense(False)
_dense_final = _make_dense(True)


@jax.jit
def kernel(x, edge_index, W_self1, W_nbr1, b1, W_self2, W_nbr2, b2,
           W_out, b_out):
    src_f = edge_index[0].astype(jnp.int32).reshape(NW, EPT)
    dst_p = edge_index[1].astype(jnp.int32).reshape(NW, CH, K)

    dall = _sc_deg(dst_p).reshape(NC, NP, 1)
    agg1 = _sc_agg(x, src_f, dst_p)

    h1 = _dense_mid(x, agg1[0], agg1[1], dall,
                    W_self1, W_nbr1, b1.reshape(1, D))

    agg2 = _sc_agg(h1, src_f, dst_p)

    out = _dense_final(h1, agg2[0], agg2[1], dall,
                       W_self2, W_nbr2, b2.reshape(1, D),
                       W_out, b_out.reshape(1, D))
    return out
